# baseline (speedup 1.0000x reference)
import math

import jax
import jax.numpy as jnp
from jax.experimental import pallas as pl
from jax.experimental.pallas import tpu as pltpu
from jax.experimental.pallas import tpu_sc as plsc

_CUTOFF = 10.0
_F32 = jnp.float32


def _silu(x):
    return x * jax.nn.sigmoid(x)


def _blk(n, cap):
    for b in range(min(cap, n), 0, -1):
        if n % b == 0 and (b % 8 == 0 or b == n):
            return b
    return n


def _mm(a, b):
    return jnp.dot(a, b, preferred_element_type=_F32)


_SC_NC = 2
_SC_NS = 16
_SC_NW = _SC_NC * _SC_NS


def _sc_gather(ta, tb, src, dst):
    n, wd = ta.shape
    e = src.shape[0]
    epw = e // _SC_NW
    ch = 80
    assert epw % ch == 0 and ch % 8 == 0
    nchunk = epw // ch
    assert nchunk % 2 == 1 and nchunk >= 5
    mesh = plsc.VectorSubcoreMesh(core_axis_name="c", subcore_axis_name="s")

    def body(a_hbm, b_hbm, src_hbm, dst_hbm, out_hbm,
             idx_s, idx_d, rows_a, rows_b, sems, wsems):
        cid = jax.lax.axis_index("c")
        sid = jax.lax.axis_index("s")
        wid = sid * _SC_NC + cid
        base = wid * epw
        pltpu.sync_copy(src_hbm.at[pl.ds(pl.multiple_of(base, 8), epw)], idx_s)
        pltpu.sync_copy(dst_hbm.at[pl.ds(pl.multiple_of(base, 8), epw)], idx_d)

        def issue(j, b):
            o = pl.ds(j * ch, ch)
            pltpu.async_copy(a_hbm.at[idx_s.at[o]], rows_a.at[b], sems.at[b])
            pltpu.async_copy(b_hbm.at[idx_d.at[o]], rows_b.at[b], sems.at[b])

        def process(j, b):
            pltpu.make_async_copy(a_hbm.at[idx_s.at[pl.ds(0, ch)]],
                                  rows_a.at[b], sems.at[b]).wait()
            pltpu.make_async_copy(b_hbm.at[idx_d.at[pl.ds(0, ch)]],
                                  rows_b.at[b], sems.at[b]).wait()

            def row(r, c2):
                for k in range(wd // 16):
                    s = pl.ds(k * 16, 16)
                    rows_a[b, r, s] = rows_a[b, r, s] + rows_b[b, r, s]
                return c2

            jax.lax.fori_loop(0, ch, row, 0, unroll=4)
            off = pl.multiple_of(base + j * ch, 8)
            pltpu.async_copy(rows_a.at[b], out_hbm.at[pl.ds(off, ch)],
                             wsems.at[b])

        def drain_w(b):
            pltpu.make_async_copy(rows_a.at[b], out_hbm.at[pl.ds(0, ch)],
                                  wsems.at[b]).wait()

        issue(0, 0)
        issue(1, 1)

        def pair(i, carry):
            j0 = i * 2
            process(j0, 0)
            drain_w(0)
            issue(j0 + 2, 0)
            process(j0 + 1, 1)
            drain_w(1)
            issue(j0 + 3, 1)
            return carry

        jax.lax.fori_loop(0, (nchunk - 3) // 2, pair, 0, unroll=False)
        process(nchunk - 3, 0)
        drain_w(0)
        issue(nchunk - 1, 0)
        process(nchunk - 2, 1)
        drain_w(1)
        process(nchunk - 1, 0)
        drain_w(0)

    f = pl.kernel(
        body,
        out_type=jax.ShapeDtypeStruct((e, wd), _F32),
        mesh=mesh,
        compiler_params=pltpu.CompilerParams(use_tc_tiling_on_sc=False),
        scratch_types=[
            pltpu.VMEM((epw,), jnp.int32),
            pltpu.VMEM((epw,), jnp.int32),
            pltpu.VMEM((2, ch, wd), _F32),
            pltpu.VMEM((2, ch, wd), _F32),
            pltpu.SemaphoreType.DMA((2,)),
            pltpu.SemaphoreType.DMA((2,)),
        ],
    )
    return f(ta, tb, src, dst)


def _sc_scatter(mw, pm, dst, n, zh, zp):
    e, hd = mw.shape
    epw = e // _SC_NW
    ch = 80
    assert epw % ch == 0
    nchunk = epw // ch
    assert nchunk % 2 == 1 and nchunk >= 3
    npt = n // _SC_NS
    assert n % _SC_NS == 0
    mesh = plsc.VectorSubcoreMesh(core_axis_name="c", subcore_axis_name="s")

    def body(mw_hbm, pm_hbm, dst2_hbm, zh_hbm, zp_hbm, agg2_hbm, dp2_hbm,
             idx2, mbuf, pbuf, acc_h, acc_p, sems):
        cid = jax.lax.axis_index("c")
        sid = jax.lax.axis_index("s")
        wid = sid * _SC_NC + cid
        base = wid * epw
        rows = pl.ds(sid * npt, npt)
        pltpu.sync_copy(zh_hbm.at[rows], acc_h.at[rows])
        pltpu.sync_copy(zp_hbm.at[rows], acc_p.at[rows])
        pltpu.sync_copy(dst2_hbm.at[pl.ds(wid * nchunk, nchunk)], idx2)
        plsc.subcore_barrier()

        def issue(j, b):
            off = pl.multiple_of(base + j * ch, 8)
            pltpu.async_copy(mw_hbm.at[pl.ds(off, ch)], mbuf.at[b], sems.at[b])
            pltpu.async_copy(pm_hbm.at[pl.ds(off, ch)], pbuf.at[b], sems.at[b])

        def process(j, b):
            pltpu.make_async_copy(mw_hbm.at[pl.ds(0, ch)], mbuf.at[b],
                                  sems.at[b]).wait()
            pltpu.make_async_copy(pm_hbm.at[pl.ds(0, ch)], pbuf.at[b],
                                  sems.at[b]).wait()
            pltpu.sync_copy(mbuf.at[b], acc_h.at[idx2.at[j]], add=True)
            pltpu.sync_copy(pbuf.at[b], acc_p.at[idx2.at[j]], add=True)

        issue(0, 0)

        def pair(i, carry):
            j0 = i * 2
            issue(j0 + 1, 1)
            process(j0, 0)
            issue(j0 + 2, 0)
            process(j0 + 1, 1)
            return carry

        jax.lax.fori_loop(0, (nchunk - 1) // 2, pair, 0, unroll=False)
        process(nchunk - 1, 0)
        plsc.subcore_barrier()

        @pl.when(cid == 0)
        def _():
            pltpu.sync_copy(acc_h.at[rows], agg2_hbm.at[0].at[rows])
            pltpu.sync_copy(acc_p.at[rows], dp2_hbm.at[0].at[rows])

        @pl.when(cid == 1)
        def _():
            pltpu.sync_copy(acc_h.at[rows], agg2_hbm.at[1].at[rows])
            pltpu.sync_copy(acc_p.at[rows], dp2_hbm.at[1].at[rows])

    f = pl.kernel(
        body,
        out_type=[
            jax.ShapeDtypeStruct((2, n, hd), _F32),
            jax.ShapeDtypeStruct((2, n, 16), _F32),
        ],
        mesh=mesh,
        compiler_params=pltpu.CompilerParams(use_tc_tiling_on_sc=False),
        scratch_types=[
            pltpu.VMEM((nchunk, ch), jnp.int32),
            pltpu.VMEM((2, ch, hd), _F32),
            pltpu.VMEM((2, ch, 16), _F32),
            pltpu.VMEM_SHARED((n, hd), _F32),
            pltpu.VMEM_SHARED((n, 16), _F32),
            pltpu.SemaphoreType.DMA((2,)),
        ],
    )
    return f(mw, pm, dst.reshape(e // ch, ch), zh, zp)


def _tables_body(h_ref, pos_ref, wa_ref, wb_ref, a_ref, b_ref):
    hv = h_ref[...]
    posv = pos_ref[...]
    hd = hv.shape[1]
    a_ref[:, :hd] = _mm(hv, wa_ref[...])
    a_ref[:, hd:] = -posv
    b_ref[:, :hd] = _mm(hv, wb_ref[...])
    b_ref[:, hd:] = posv


def _tables_call(h, pos16, wa, wb):
    n, hd = h.shape
    wd = hd + 16
    nb = _blk(n, 2500)
    grid = (n // nb,)
    return pl.pallas_call(
        _tables_body,
        grid=grid,
        in_specs=[
            pl.BlockSpec((nb, hd), lambda i: (i, 0)),
            pl.BlockSpec((nb, 16), lambda i: (i, 0)),
            pl.BlockSpec((hd, hd), lambda i: (0, 0)),
            pl.BlockSpec((hd, hd), lambda i: (0, 0)),
        ],
        out_specs=[
            pl.BlockSpec((nb, wd), lambda i: (i, 0)),
            pl.BlockSpec((nb, wd), lambda i: (i, 0)),
        ],
        out_shape=[
            jax.ShapeDtypeStruct((n, wd), _F32),
            jax.ShapeDtypeStruct((n, wd), _F32),
        ],
    )(h, pos16, wa, wb)


def _edge_body(nfreq, hd, er_ref, w1c_ref, b1_ref, w2_ref, b2_ref,
               cw1_ref, cb1_ref, cw2_ref, mw_ref, pm_ref, loss_ref):
    rel = er_ref[:, hd:]
    d2 = jnp.sum(rel * rel, axis=1, keepdims=True) + 1e-8
    dist = jnp.sqrt(d2)
    dist_t = jnp.transpose(dist)
    freqs_c = ((jax.lax.broadcasted_iota(jnp.int32, (nfreq, 1), 0).astype(_F32))
               + 1.0) * (math.pi / _CUTOFF)
    demb_t = jnp.sin(freqs_c * dist_t)
    pre = er_ref[:, :hd] + jax.lax.dot_general(
        demb_t, w1c_ref[...], (((0,), (0,)), ((), ())),
        preferred_element_type=_F32) + b1_ref[...]
    m = _silu(pre)
    m = _silu(_mm(m, w2_ref[...]) + b2_ref[...])
    w_t = (0.5 * (jnp.cos(dist_t * (math.pi / _CUTOFF)) + 1.0)
           * (dist_t < _CUTOFF).astype(_F32))
    w = jnp.transpose(w_t)
    mw = m * w
    ch = _silu(_mm(mw, cw1_ref[...]) + cb1_ref[...])
    cs = jnp.tanh(_mm(ch, cw2_ref[...]))
    mw_ref[...] = mw
    pm_ref[...] = rel * (cs / (dist + 1.0))
    part = jnp.sum((dist_t - 1.5) ** 2 * w_t)

    @pl.when(pl.program_id(0) == 0)
    def _():
        loss_ref[0, 0] = part

    @pl.when(pl.program_id(0) != 0)
    def _():
        loss_ref[0, 0] += part


def _edge_call(er, w1c, b1, w2, b2, cw1, cb1, cw2):
    e, wd = er.shape
    hd = wd - 16
    nfreq = w1c.shape[0]
    eb = _blk(e, 2500)
    grid = (e // eb,)
    import functools
    return pl.pallas_call(
        functools.partial(_edge_body, nfreq, hd),
        grid=grid,
        in_specs=[
            pl.BlockSpec((eb, wd), lambda i: (i, 0)),
            pl.BlockSpec((nfreq, hd), lambda i: (0, 0)),
            pl.BlockSpec((1, hd), lambda i: (0, 0)),
            pl.BlockSpec((hd, hd), lambda i: (0, 0)),
            pl.BlockSpec((1, hd), lambda i: (0, 0)),
            pl.BlockSpec((hd, hd), lambda i: (0, 0)),
            pl.BlockSpec((1, hd), lambda i: (0, 0)),
            pl.BlockSpec((hd, 1), lambda i: (0, 0)),
        ],
        out_specs=[
            pl.BlockSpec((eb, hd), lambda i: (i, 0)),
            pl.BlockSpec((eb, 16), lambda i: (i, 0)),
            pl.BlockSpec(memory_space=pltpu.SMEM),
        ],
        out_shape=[
            jax.ShapeDtypeStruct((e, hd), _F32),
            jax.ShapeDtypeStruct((e, 16), _F32),
            jax.ShapeDtypeStruct((1, 1), _F32),
        ],
    )(er, w1c, b1, w2, b2, cw1, cb1, cw2)


def _node_body(h_ref, agga_ref, aggb_ref, dpa_ref, dpb_ref, pos_ref,
               nw1a_ref, nw1b_ref, nb1_ref, nw2_ref, nb2_ref, hn_ref, pn_ref):
    agg = agga_ref[...] + aggb_ref[...]
    upd = _silu(_mm(h_ref[...], nw1a_ref[...]) + _mm(agg, nw1b_ref[...]) + nb1_ref[...])
    hn_ref[...] = h_ref[...] + _mm(upd, nw2_ref[...]) + nb2_ref[...]
    pn_ref[...] = pos_ref[...] + dpa_ref[...] + dpb_ref[...]


def _node_call(h, agga, aggb, dpa, dpb, pos16, nw1a, nw1b, nb1, nw2, nb2):
    n, hd = h.shape
    nb = _blk(n, 2500)
    grid = (n // nb,)
    return pl.pallas_call(
        _node_body,
        grid=grid,
        in_specs=[
            pl.BlockSpec((nb, hd), lambda i: (i, 0)),
            pl.BlockSpec((nb, hd), lambda i: (i, 0)),
            pl.BlockSpec((nb, hd), lambda i: (i, 0)),
            pl.BlockSpec((nb, 16), lambda i: (i, 0)),
            pl.BlockSpec((nb, 16), lambda i: (i, 0)),
            pl.BlockSpec((nb, 16), lambda i: (i, 0)),
            pl.BlockSpec((hd, hd), lambda i: (0, 0)),
            pl.BlockSpec((hd, hd), lambda i: (0, 0)),
            pl.BlockSpec((1, hd), lambda i: (0, 0)),
            pl.BlockSpec((hd, hd), lambda i: (0, 0)),
            pl.BlockSpec((1, hd), lambda i: (0, 0)),
        ],
        out_specs=[
            pl.BlockSpec((nb, hd), lambda i: (i, 0)),
            pl.BlockSpec((nb, 16), lambda i: (i, 0)),
        ],
        out_shape=[
            jax.ShapeDtypeStruct((n, hd), _F32),
            jax.ShapeDtypeStruct((n, 16), _F32),
        ],
    )(h, agga, aggb, dpa, dpb, pos16, nw1a, nw1b, nb1, nw2, nb2)


def _colsum_body(pos_ref, out_ref):
    s = jnp.sum(pos_ref[...], axis=0, keepdims=True)

    @pl.when(pl.program_id(0) == 0)
    def _():
        out_ref[...] = s

    @pl.when(pl.program_id(0) != 0)
    def _():
        out_ref[...] += s


def _colsum_call(pos16):
    n, _ = pos16.shape
    nb = _blk(n, 2500)
    return pl.pallas_call(
        _colsum_body,
        grid=(n // nb,),
        in_specs=[pl.BlockSpec((nb, 16), lambda i: (i, 0))],
        out_specs=pl.BlockSpec((1, 16), lambda i: (0, 0)),
        out_shape=jax.ShapeDtypeStruct((1, 16), _F32),
    )(pos16)


def _heads_body(n_nodes, h_ref, pos_ref, csum_ref, gw1_ref, gb1_ref, gw2_ref,
                gb2_ref, iw1a_ref, iw1b0_ref, wmean_ref, ib1_ref, iw2_ref,
                ib2_ref, geo_ref, inv_ref):
    hv = h_ref[...]
    geo_ref[...] = _mm(_silu(_mm(hv, gw1_ref[...]) + gb1_ref[...]), gw2_ref[...]) + gb2_ref[...]
    posv = pos_ref[...]
    norm = jnp.sqrt(jnp.sum(posv * posv, axis=1, keepdims=True))
    mean16 = csum_ref[...] * (1.0 / n_nodes)
    crow = _mm(mean16, wmean_ref[...])
    pre = _mm(hv, iw1a_ref[...]) + norm * iw1b0_ref[...] + crow + ib1_ref[...]
    inv_ref[...] = _mm(_silu(pre), iw2_ref[...]) + ib2_ref[...]


def _heads_call(h, pos16, csum, gw1, gb1, gw2, gb2, iw1a, iw1b0, wmean, ib1,
                iw2, ib2):
    n, hd = h.shape
    go = gw2.shape[1]
    nb = _blk(n, 2500)
    import functools
    return pl.pallas_call(
        functools.partial(_heads_body, n),
        grid=(n // nb,),
        in_specs=[
            pl.BlockSpec((nb, hd), lambda i: (i, 0)),
            pl.BlockSpec((nb, 16), lambda i: (i, 0)),
            pl.BlockSpec((1, 16), lambda i: (0, 0)),
            pl.BlockSpec((hd, gw1.shape[1]), lambda i: (0, 0)),
            pl.BlockSpec((1, gb1.shape[1]), lambda i: (0, 0)),
            pl.BlockSpec((gw2.shape[0], go), lambda i: (0, 0)),
            pl.BlockSpec((1, go), lambda i: (0, 0)),
            pl.BlockSpec((hd, hd), lambda i: (0, 0)),
            pl.BlockSpec((1, hd), lambda i: (0, 0)),
            pl.BlockSpec((16, hd), lambda i: (0, 0)),
            pl.BlockSpec((1, hd), lambda i: (0, 0)),
            pl.BlockSpec((hd, hd), lambda i: (0, 0)),
            pl.BlockSpec((1, hd), lambda i: (0, 0)),
        ],
        out_specs=[
            pl.BlockSpec((nb, go), lambda i: (i, 0)),
            pl.BlockSpec((nb, hd), lambda i: (i, 0)),
        ],
        out_shape=[
            jax.ShapeDtypeStruct((n, go), _F32),
            jax.ShapeDtypeStruct((n, hd), _F32),
        ],
    )(h, pos16, csum, gw1, gb1, gw2, gb2, iw1a, iw1b0, wmean, ib1, iw2, ib2)


def kernel(h, pos, batch, edge_index, params):
    n, hd = h.shape
    e = edge_index.shape[1]
    nl = 0
    while f"e{nl}_W1" in params:
        nl += 1
    src = edge_index[0]
    dst = edge_index[1]
    pos16 = jnp.zeros((n, 16), _F32).at[:, :3].set(pos)

    def b2d(v):
        return v.reshape(1, -1)

    loss_parts = []
    for l in range(nl):
        w1 = params[f"e{l}_W1"]
        ta, tb = _tables_call(h, pos16, w1[:hd], w1[hd:2 * hd])
        er = _sc_gather(ta, tb, src, dst)
        mw, pm, lp = _edge_call(
            er, w1[2 * hd:], b2d(params[f"e{l}_b1"]),
            params[f"e{l}_W2"], b2d(params[f"e{l}_b2"]),
            params[f"c{l}_W1"], b2d(params[f"c{l}_b1"]), params[f"c{l}_W2"])
        zh = jnp.zeros((n, hd), _F32)
        zp = jnp.zeros((n, 16), _F32)
        agg2, dp2 = _sc_scatter(mw, pm, dst, n, zh, zp)
        nw1 = params[f"n{l}_W1"]
        h, pos16 = _node_call(
            h, agg2[0], agg2[1], dp2[0], dp2[1], pos16, nw1[:hd], nw1[hd:],
            b2d(params[f"n{l}_b1"]), params[f"n{l}_W2"], b2d(params[f"n{l}_b2"]))
        loss_parts.append(lp[0, 0])

    csum = _colsum_call(pos16)
    iw1 = params["i_W1"]
    wmean = jnp.zeros((16, hd), _F32).at[:3].set(iw1[hd + 1:hd + 4])
    geo, inv = _heads_call(
        h, pos16, csum, params["g_W1"], b2d(params["g_b1"]), params["g_W2"],
        b2d(params["g_b2"]), iw1[:hd], iw1[hd:hd + 1], wmean,
        b2d(params["i_b1"]), params["i_W2"], b2d(params["i_b2"]))

    closs = sum(loss_parts) / e
    return (h, pos16[:, :3], geo, inv, closs)


# speedup vs baseline: 1.1759x; 1.1759x over previous
import math

import jax
import jax.numpy as jnp
from jax.experimental import pallas as pl
from jax.experimental.pallas import tpu as pltpu
from jax.experimental.pallas import tpu_sc as plsc

_CUTOFF = 10.0
_F32 = jnp.float32


def _silu(x):
    return x * jax.nn.sigmoid(x)


def _blk(n, cap):
    for b in range(min(cap, n), 0, -1):
        if n % b == 0 and (b % 8 == 0 or b == n):
            return b
    return n


def _mm(a, b):
    return jnp.dot(a, b, preferred_element_type=_F32)


_SC_NC = 2
_SC_NS = 16
_SC_NW = _SC_NC * _SC_NS


def _sc_gather(ta, tb, pos16, src, dst):
    n, hd = ta.shape
    e = src.shape[0]
    epw = e // _SC_NW
    ch = 80
    assert epw % ch == 0 and ch % 8 == 0
    nchunk = epw // ch
    assert nchunk % 2 == 1 and nchunk >= 5
    mesh = plsc.VectorSubcoreMesh(core_axis_name="c", subcore_axis_name="s")

    def body(a_hbm, b_hbm, p_hbm, src_hbm, dst_hbm, esum_hbm, relp_hbm,
             idx_s, idx_d, rows_a, rows_b, pa, pb, sems, wsems):
        cid = jax.lax.axis_index("c")
        sid = jax.lax.axis_index("s")
        wid = sid * _SC_NC + cid
        base = wid * epw
        pltpu.sync_copy(src_hbm.at[pl.ds(pl.multiple_of(base, 8), epw)], idx_s)
        pltpu.sync_copy(dst_hbm.at[pl.ds(pl.multiple_of(base, 8), epw)], idx_d)

        def issue(j, b):
            o = pl.ds(j * ch, ch)
            pltpu.async_copy(a_hbm.at[idx_s.at[o]], rows_a.at[b], sems.at[b])
            pltpu.async_copy(b_hbm.at[idx_d.at[o]], rows_b.at[b], sems.at[b])
            pltpu.async_copy(p_hbm.at[idx_s.at[o]], pa.at[b], sems.at[b])
            pltpu.async_copy(p_hbm.at[idx_d.at[o]], pb.at[b], sems.at[b])

        def process(j, b):
            pltpu.make_async_copy(a_hbm.at[idx_s.at[pl.ds(0, ch)]],
                                  rows_a.at[b], sems.at[b]).wait()
            pltpu.make_async_copy(b_hbm.at[idx_d.at[pl.ds(0, ch)]],
                                  rows_b.at[b], sems.at[b]).wait()
            pltpu.make_async_copy(p_hbm.at[idx_s.at[pl.ds(0, ch)]],
                                  pa.at[b], sems.at[b]).wait()
            pltpu.make_async_copy(p_hbm.at[idx_d.at[pl.ds(0, ch)]],
                                  pb.at[b], sems.at[b]).wait()

            def row(r, c2):
                for k in range(hd // 16):
                    s = pl.ds(k * 16, 16)
                    rows_a[b, r, s] = rows_a[b, r, s] + rows_b[b, r, s]
                pb[b, r, :] = pb[b, r, :] - pa[b, r, :]
                return c2

            jax.lax.fori_loop(0, ch, row, 0, unroll=4)
            off = pl.multiple_of(base + j * ch, 8)
            pltpu.async_copy(rows_a.at[b], esum_hbm.at[pl.ds(off, ch)],
                             wsems.at[b])
            pltpu.async_copy(pb.at[b], relp_hbm.at[pl.ds(off, ch)],
                             wsems.at[b])

        def drain_w(b):
            pltpu.make_async_copy(rows_a.at[b], esum_hbm.at[pl.ds(0, ch)],
                                  wsems.at[b]).wait()
            pltpu.make_async_copy(pb.at[b], relp_hbm.at[pl.ds(0, ch)],
                                  wsems.at[b]).wait()

        issue(0, 0)
        issue(1, 1)

        def pair(i, carry):
            j0 = i * 2
            process(j0, 0)
            drain_w(0)
            issue(j0 + 2, 0)
            process(j0 + 1, 1)
            drain_w(1)
            issue(j0 + 3, 1)
            return carry

        jax.lax.fori_loop(0, (nchunk - 3) // 2, pair, 0, unroll=False)
        process(nchunk - 3, 0)
        drain_w(0)
        issue(nchunk - 1, 0)
        process(nchunk - 2, 1)
        drain_w(1)
        process(nchunk - 1, 0)
        drain_w(0)

    f = pl.kernel(
        body,
        out_type=[
            jax.ShapeDtypeStruct((e, hd), _F32),
            jax.ShapeDtypeStruct((e, 16), _F32),
        ],
        mesh=mesh,
        compiler_params=pltpu.CompilerParams(use_tc_tiling_on_sc=False),
        scratch_types=[
            pltpu.VMEM((epw,), jnp.int32),
            pltpu.VMEM((epw,), jnp.int32),
            pltpu.VMEM((2, ch, hd), _F32),
            pltpu.VMEM((2, ch, hd), _F32),
            pltpu.VMEM((2, ch, 16), _F32),
            pltpu.VMEM((2, ch, 16), _F32),
            pltpu.SemaphoreType.DMA((2,)),
            pltpu.SemaphoreType.DMA((2,)),
        ],
    )
    return f(ta, tb, pos16, src, dst)


def _sc_scatter(mw, pm, dst, n, zh, zp):
    e, hd = mw.shape
    epw = e // _SC_NW
    ch = 80
    assert epw % ch == 0
    nchunk = epw // ch
    assert nchunk % 2 == 1 and nchunk >= 3
    npt = n // _SC_NS
    assert n % _SC_NS == 0
    mesh = plsc.VectorSubcoreMesh(core_axis_name="c", subcore_axis_name="s")

    def body(mw_hbm, pm_hbm, dst2_hbm, zh_hbm, zp_hbm, agg2_hbm, dp2_hbm,
             idx2, mbuf, pbuf, acc_h, acc_p, sems):
        cid = jax.lax.axis_index("c")
        sid = jax.lax.axis_index("s")
        wid = sid * _SC_NC + cid
        base = wid * epw
        rows = pl.ds(sid * npt, npt)
        pltpu.sync_copy(zh_hbm.at[rows], acc_h.at[rows])
        pltpu.sync_copy(zp_hbm.at[rows], acc_p.at[rows])
        pltpu.sync_copy(dst2_hbm.at[pl.ds(wid * nchunk, nchunk)], idx2)
        plsc.subcore_barrier()

        def issue(j, b):
            off = pl.multiple_of(base + j * ch, 8)
            pltpu.async_copy(mw_hbm.at[pl.ds(off, ch)], mbuf.at[b], sems.at[b])
            pltpu.async_copy(pm_hbm.at[pl.ds(off, ch)], pbuf.at[b], sems.at[b])

        def process(j, b):
            pltpu.make_async_copy(mw_hbm.at[pl.ds(0, ch)], mbuf.at[b],
                                  sems.at[b]).wait()
            pltpu.make_async_copy(pm_hbm.at[pl.ds(0, ch)], pbuf.at[b],
                                  sems.at[b]).wait()
            pltpu.sync_copy(mbuf.at[b], acc_h.at[idx2.at[j]], add=True)
            pltpu.sync_copy(pbuf.at[b], acc_p.at[idx2.at[j]], add=True)

        issue(0, 0)

        def pair(i, carry):
            j0 = i * 2
            issue(j0 + 1, 1)
            process(j0, 0)
            issue(j0 + 2, 0)
            process(j0 + 1, 1)
            return carry

        jax.lax.fori_loop(0, (nchunk - 1) // 2, pair, 0, unroll=False)
        process(nchunk - 1, 0)
        plsc.subcore_barrier()

        @pl.when(cid == 0)
        def _():
            pltpu.sync_copy(acc_h.at[rows], agg2_hbm.at[0].at[rows])
            pltpu.sync_copy(acc_p.at[rows], dp2_hbm.at[0].at[rows])

        @pl.when(cid == 1)
        def _():
            pltpu.sync_copy(acc_h.at[rows], agg2_hbm.at[1].at[rows])
            pltpu.sync_copy(acc_p.at[rows], dp2_hbm.at[1].at[rows])

    f = pl.kernel(
        body,
        out_type=[
            jax.ShapeDtypeStruct((2, n, hd), _F32),
            jax.ShapeDtypeStruct((2, n, 16), _F32),
        ],
        mesh=mesh,
        compiler_params=pltpu.CompilerParams(use_tc_tiling_on_sc=False),
        scratch_types=[
            pltpu.VMEM((nchunk, ch), jnp.int32),
            pltpu.VMEM((2, ch, hd), _F32),
            pltpu.VMEM((2, ch, 16), _F32),
            pltpu.VMEM_SHARED((n, hd), _F32),
            pltpu.VMEM_SHARED((n, 16), _F32),
            pltpu.SemaphoreType.DMA((2,)),
        ],
    )
    return f(mw, pm, dst.reshape(e // ch, ch), zh, zp)


def _tables_body(h_ref, wa_ref, wb_ref, a_ref, b_ref):
    hv = h_ref[...]
    a_ref[...] = _mm(hv, wa_ref[...])
    b_ref[...] = _mm(hv, wb_ref[...])


def _tables_call(h, wa, wb):
    n, hd = h.shape
    nb = _blk(n, 2500)
    grid = (n // nb,)
    return pl.pallas_call(
        _tables_body,
        grid=grid,
        in_specs=[
            pl.BlockSpec((nb, hd), lambda i: (i, 0)),
            pl.BlockSpec((hd, hd), lambda i: (0, 0)),
            pl.BlockSpec((hd, hd), lambda i: (0, 0)),
        ],
        out_specs=[
            pl.BlockSpec((nb, hd), lambda i: (i, 0)),
            pl.BlockSpec((nb, hd), lambda i: (i, 0)),
        ],
        out_shape=[
            jax.ShapeDtypeStruct((n, hd), _F32),
            jax.ShapeDtypeStruct((n, hd), _F32),
        ],
    )(h, wa, wb)


def _edge_body(nfreq, esum_ref, relp_ref, w1c_ref, b1_ref, w2_ref, b2_ref,
               cw1_ref, cb1_ref, cw2_ref, mw_ref, pm_ref, loss_ref):
    rel = relp_ref[...]
    d2 = jnp.sum(rel * rel, axis=1, keepdims=True) + 1e-8
    dist = jnp.sqrt(d2)
    dist_t = jnp.transpose(dist)
    freqs_c = ((jax.lax.broadcasted_iota(jnp.int32, (nfreq, 1), 0).astype(_F32))
               + 1.0) * (math.pi / _CUTOFF)
    demb_t = jnp.sin(freqs_c * dist_t)
    pre = esum_ref[...] + jax.lax.dot_general(
        demb_t, w1c_ref[...], (((0,), (0,)), ((), ())),
        preferred_element_type=_F32) + b1_ref[...]
    m = _silu(pre)
    m = _silu(_mm(m, w2_ref[...]) + b2_ref[...])
    w_t = (0.5 * (jnp.cos(dist_t * (math.pi / _CUTOFF)) + 1.0)
           * (dist_t < _CUTOFF).astype(_F32))
    w = jnp.transpose(w_t)
    mw = m * w
    ch = _silu(_mm(mw, cw1_ref[...]) + cb1_ref[...])
    cs = jnp.tanh(_mm(ch, cw2_ref[...]))
    mw_ref[...] = mw
    pm_ref[...] = rel * (cs / (dist + 1.0))
    part = jnp.sum((dist_t - 1.5) ** 2 * w_t)

    @pl.when(pl.program_id(0) == 0)
    def _():
        loss_ref[0, 0] = part

    @pl.when(pl.program_id(0) != 0)
    def _():
        loss_ref[0, 0] += part


def _edge_call(esum, relp, w1c, b1, w2, b2, cw1, cb1, cw2):
    e, hd = esum.shape
    nfreq = w1c.shape[0]
    eb = _blk(e, 2500)
    grid = (e // eb,)
    import functools
    return pl.pallas_call(
        functools.partial(_edge_body, nfreq),
        grid=grid,
        in_specs=[
            pl.BlockSpec((eb, hd), lambda i: (i, 0)),
            pl.BlockSpec((eb, 16), lambda i: (i, 0)),
            pl.BlockSpec((nfreq, hd), lambda i: (0, 0)),
            pl.BlockSpec((1, hd), lambda i: (0, 0)),
            pl.BlockSpec((hd, hd), lambda i: (0, 0)),
            pl.BlockSpec((1, hd), lambda i: (0, 0)),
            pl.BlockSpec((hd, hd), lambda i: (0, 0)),
            pl.BlockSpec((1, hd), lambda i: (0, 0)),
            pl.BlockSpec((hd, 1), lambda i: (0, 0)),
        ],
        out_specs=[
            pl.BlockSpec((eb, hd), lambda i: (i, 0)),
            pl.BlockSpec((eb, 16), lambda i: (i, 0)),
            pl.BlockSpec(memory_space=pltpu.SMEM),
        ],
        out_shape=[
            jax.ShapeDtypeStruct((e, hd), _F32),
            jax.ShapeDtypeStruct((e, 16), _F32),
            jax.ShapeDtypeStruct((1, 1), _F32),
        ],
    )(esum, relp, w1c, b1, w2, b2, cw1, cb1, cw2)


def _node_body(h_ref, agga_ref, aggb_ref, dpa_ref, dpb_ref, pos_ref,
               nw1a_ref, nw1b_ref, nb1_ref, nw2_ref, nb2_ref, hn_ref, pn_ref):
    agg = agga_ref[...] + aggb_ref[...]
    upd = _silu(_mm(h_ref[...], nw1a_ref[...]) + _mm(agg, nw1b_ref[...]) + nb1_ref[...])
    hn_ref[...] = h_ref[...] + _mm(upd, nw2_ref[...]) + nb2_ref[...]
    pn_ref[...] = pos_ref[...] + dpa_ref[...] + dpb_ref[...]


def _node_call(h, agga, aggb, dpa, dpb, pos16, nw1a, nw1b, nb1, nw2, nb2):
    n, hd = h.shape
    nb = _blk(n, 2500)
    grid = (n // nb,)
    return pl.pallas_call(
        _node_body,
        grid=grid,
        in_specs=[
            pl.BlockSpec((nb, hd), lambda i: (i, 0)),
            pl.BlockSpec((nb, hd), lambda i: (i, 0)),
            pl.BlockSpec((nb, hd), lambda i: (i, 0)),
            pl.BlockSpec((nb, 16), lambda i: (i, 0)),
            pl.BlockSpec((nb, 16), lambda i: (i, 0)),
            pl.BlockSpec((nb, 16), lambda i: (i, 0)),
            pl.BlockSpec((hd, hd), lambda i: (0, 0)),
            pl.BlockSpec((hd, hd), lambda i: (0, 0)),
            pl.BlockSpec((1, hd), lambda i: (0, 0)),
            pl.BlockSpec((hd, hd), lambda i: (0, 0)),
            pl.BlockSpec((1, hd), lambda i: (0, 0)),
        ],
        out_specs=[
            pl.BlockSpec((nb, hd), lambda i: (i, 0)),
            pl.BlockSpec((nb, 16), lambda i: (i, 0)),
        ],
        out_shape=[
            jax.ShapeDtypeStruct((n, hd), _F32),
            jax.ShapeDtypeStruct((n, 16), _F32),
        ],
    )(h, agga, aggb, dpa, dpb, pos16, nw1a, nw1b, nb1, nw2, nb2)


def _colsum_body(pos_ref, out_ref):
    s = jnp.sum(pos_ref[...], axis=0, keepdims=True)

    @pl.when(pl.program_id(0) == 0)
    def _():
        out_ref[...] = s

    @pl.when(pl.program_id(0) != 0)
    def _():
        out_ref[...] += s


def _colsum_call(pos16):
    n, _ = pos16.shape
    nb = _blk(n, 2500)
    return pl.pallas_call(
        _colsum_body,
        grid=(n // nb,),
        in_specs=[pl.BlockSpec((nb, 16), lambda i: (i, 0))],
        out_specs=pl.BlockSpec((1, 16), lambda i: (0, 0)),
        out_shape=jax.ShapeDtypeStruct((1, 16), _F32),
    )(pos16)


def _heads_body(n_nodes, h_ref, pos_ref, csum_ref, gw1_ref, gb1_ref, gw2_ref,
                gb2_ref, iw1a_ref, iw1b0_ref, wmean_ref, ib1_ref, iw2_ref,
                ib2_ref, geo_ref, inv_ref):
    hv = h_ref[...]
    geo_ref[...] = _mm(_silu(_mm(hv, gw1_ref[...]) + gb1_ref[...]), gw2_ref[...]) + gb2_ref[...]
    posv = pos_ref[...]
    norm = jnp.sqrt(jnp.sum(posv * posv, axis=1, keepdims=True))
    mean16 = csum_ref[...] * (1.0 / n_nodes)
    crow = _mm(mean16, wmean_ref[...])
    pre = _mm(hv, iw1a_ref[...]) + norm * iw1b0_ref[...] + crow + ib1_ref[...]
    inv_ref[...] = _mm(_silu(pre), iw2_ref[...]) + ib2_ref[...]


def _heads_call(h, pos16, csum, gw1, gb1, gw2, gb2, iw1a, iw1b0, wmean, ib1,
                iw2, ib2):
    n, hd = h.shape
    go = gw2.shape[1]
    nb = _blk(n, 2500)
    import functools
    return pl.pallas_call(
        functools.partial(_heads_body, n),
        grid=(n // nb,),
        in_specs=[
            pl.BlockSpec((nb, hd), lambda i: (i, 0)),
            pl.BlockSpec((nb, 16), lambda i: (i, 0)),
            pl.BlockSpec((1, 16), lambda i: (0, 0)),
            pl.BlockSpec((hd, gw1.shape[1]), lambda i: (0, 0)),
            pl.BlockSpec((1, gb1.shape[1]), lambda i: (0, 0)),
            pl.BlockSpec((gw2.shape[0], go), lambda i: (0, 0)),
            pl.BlockSpec((1, go), lambda i: (0, 0)),
            pl.BlockSpec((hd, hd), lambda i: (0, 0)),
            pl.BlockSpec((1, hd), lambda i: (0, 0)),
            pl.BlockSpec((16, hd), lambda i: (0, 0)),
            pl.BlockSpec((1, hd), lambda i: (0, 0)),
            pl.BlockSpec((hd, hd), lambda i: (0, 0)),
            pl.BlockSpec((1, hd), lambda i: (0, 0)),
        ],
        out_specs=[
            pl.BlockSpec((nb, go), lambda i: (i, 0)),
            pl.BlockSpec((nb, hd), lambda i: (i, 0)),
        ],
        out_shape=[
            jax.ShapeDtypeStruct((n, go), _F32),
            jax.ShapeDtypeStruct((n, hd), _F32),
        ],
    )(h, pos16, csum, gw1, gb1, gw2, gb2, iw1a, iw1b0, wmean, ib1, iw2, ib2)


def kernel(h, pos, batch, edge_index, params):
    n, hd = h.shape
    e = edge_index.shape[1]
    nl = 0
    while f"e{nl}_W1" in params:
        nl += 1
    src = edge_index[0]
    dst = edge_index[1]
    pos16 = jnp.zeros((n, 16), _F32).at[:, :3].set(pos)

    def b2d(v):
        return v.reshape(1, -1)

    loss_parts = []
    for l in range(nl):
        w1 = params[f"e{l}_W1"]
        ta, tb = _tables_call(h, w1[:hd], w1[hd:2 * hd])
        esum, relp = _sc_gather(ta, tb, pos16, src, dst)
        mw, pm, lp = _edge_call(
            esum, relp, w1[2 * hd:], b2d(params[f"e{l}_b1"]),
            params[f"e{l}_W2"], b2d(params[f"e{l}_b2"]),
            params[f"c{l}_W1"], b2d(params[f"c{l}_b1"]), params[f"c{l}_W2"])
        zh = jnp.zeros((n, hd), _F32)
        zp = jnp.zeros((n, 16), _F32)
        agg2, dp2 = _sc_scatter(mw, pm, dst, n, zh, zp)
        nw1 = params[f"n{l}_W1"]
        h, pos16 = _node_call(
            h, agg2[0], agg2[1], dp2[0], dp2[1], pos16, nw1[:hd], nw1[hd:],
            b2d(params[f"n{l}_b1"]), params[f"n{l}_W2"], b2d(params[f"n{l}_b2"]))
        loss_parts.append(lp[0, 0])

    csum = _colsum_call(pos16)
    iw1 = params["i_W1"]
    wmean = jnp.zeros((16, hd), _F32).at[:3].set(iw1[hd + 1:hd + 4])
    geo, inv = _heads_call(
        h, pos16, csum, params["g_W1"], b2d(params["g_b1"]), params["g_W2"],
        b2d(params["g_b2"]), iw1[:hd], iw1[hd:hd + 1], wmean,
        b2d(params["i_b1"]), params["i_W2"], b2d(params["i_b2"]))

    closs = sum(loss_parts) / e
    return (h, pos16[:, :3], geo, inv, closs)


# speedup vs baseline: 1.4554x; 1.2377x over previous
import math

import jax
import jax.numpy as jnp
from jax.experimental import pallas as pl
from jax.experimental.pallas import tpu as pltpu
from jax.experimental.pallas import tpu_sc as plsc

_CUTOFF = 10.0
_F32 = jnp.float32


def _silu(x):
    return x * jax.nn.sigmoid(x)


def _blk(n, cap):
    for b in range(min(cap, n), 0, -1):
        if n % b == 0 and (b % 8 == 0 or b == n):
            return b
    return n


def _mm(a, b):
    return jnp.dot(a, b, preferred_element_type=_F32)


_SC_NC = 2
_SC_NS = 16
_SC_NW = _SC_NC * _SC_NS


def _sc_ch(epw):
    for c in range(128, 7, -8):
        if epw % c == 0 and (epw // c) % 2 == 1 and epw // c >= 5:
            return c
    raise ValueError(f"no chunk size for {epw}")


def _sc_gather(ta, tb, pos16, src, dst):
    n, hd = ta.shape
    e = src.shape[0]
    epw = e // _SC_NW
    ch = _sc_ch(epw)
    nchunk = epw // ch
    mesh = plsc.VectorSubcoreMesh(core_axis_name="c", subcore_axis_name="s")

    def body(a_hbm, b_hbm, p_hbm, src_hbm, dst_hbm, esum_hbm, relp_hbm,
             idx_s, idx_d, rows_a, rows_b, pa, pb, sems, wsems):
        cid = jax.lax.axis_index("c")
        sid = jax.lax.axis_index("s")
        wid = sid * _SC_NC + cid
        base = wid * epw
        pltpu.sync_copy(src_hbm.at[pl.ds(pl.multiple_of(base, 8), epw)], idx_s)
        pltpu.sync_copy(dst_hbm.at[pl.ds(pl.multiple_of(base, 8), epw)], idx_d)

        def issue(j, b):
            o = pl.ds(j * ch, ch)
            pltpu.async_copy(a_hbm.at[idx_s.at[o]], rows_a.at[b], sems.at[b])
            pltpu.async_copy(b_hbm.at[idx_d.at[o]], rows_b.at[b], sems.at[b])
            pltpu.async_copy(p_hbm.at[idx_s.at[o]], pa.at[b], sems.at[b])
            pltpu.async_copy(p_hbm.at[idx_d.at[o]], pb.at[b], sems.at[b])

        def process(j, b):
            pltpu.make_async_copy(a_hbm.at[idx_s.at[pl.ds(0, ch)]],
                                  rows_a.at[b], sems.at[b]).wait()
            pltpu.make_async_copy(b_hbm.at[idx_d.at[pl.ds(0, ch)]],
                                  rows_b.at[b], sems.at[b]).wait()
            pltpu.make_async_copy(p_hbm.at[idx_s.at[pl.ds(0, ch)]],
                                  pa.at[b], sems.at[b]).wait()
            pltpu.make_async_copy(p_hbm.at[idx_d.at[pl.ds(0, ch)]],
                                  pb.at[b], sems.at[b]).wait()

            def row(r, c2):
                for k in range(hd // 16):
                    s = pl.ds(k * 16, 16)
                    rows_a[b, r, s] = rows_a[b, r, s] + rows_b[b, r, s]
                pb[b, r, :] = pb[b, r, :] - pa[b, r, :]
                return c2

            jax.lax.fori_loop(0, ch, row, 0, unroll=4)
            off = pl.multiple_of(base + j * ch, 8)
            pltpu.async_copy(rows_a.at[b], esum_hbm.at[pl.ds(off, ch)],
                             wsems.at[b])
            pltpu.async_copy(pb.at[b], relp_hbm.at[pl.ds(off, ch)],
                             wsems.at[b])

        def drain_w(b):
            pltpu.make_async_copy(rows_a.at[b], esum_hbm.at[pl.ds(0, ch)],
                                  wsems.at[b]).wait()
            pltpu.make_async_copy(pb.at[b], relp_hbm.at[pl.ds(0, ch)],
                                  wsems.at[b]).wait()

        issue(0, 0)
        issue(1, 1)

        def pair(i, carry):
            j0 = i * 2
            process(j0, 0)
            drain_w(0)
            issue(j0 + 2, 0)
            process(j0 + 1, 1)
            drain_w(1)
            issue(j0 + 3, 1)
            return carry

        jax.lax.fori_loop(0, (nchunk - 3) // 2, pair, 0, unroll=False)
        process(nchunk - 3, 0)
        drain_w(0)
        issue(nchunk - 1, 0)
        process(nchunk - 2, 1)
        drain_w(1)
        process(nchunk - 1, 0)
        drain_w(0)

    f = pl.kernel(
        body,
        out_type=[
            jax.ShapeDtypeStruct((e, hd), _F32),
            jax.ShapeDtypeStruct((e, 16), _F32),
        ],
        mesh=mesh,
        compiler_params=pltpu.CompilerParams(use_tc_tiling_on_sc=False),
        scratch_types=[
            pltpu.VMEM((epw,), jnp.int32),
            pltpu.VMEM((epw,), jnp.int32),
            pltpu.VMEM((2, ch, hd), _F32),
            pltpu.VMEM((2, ch, hd), _F32),
            pltpu.VMEM((2, ch, 16), _F32),
            pltpu.VMEM((2, ch, 16), _F32),
            pltpu.SemaphoreType.DMA((2,)),
            pltpu.SemaphoreType.DMA((2,)),
        ],
    )
    return f(ta, tb, pos16, src, dst)


def _sc_scatter(mw, pm, dst, n, zh, zp):
    e, hd = mw.shape
    epw = e // _SC_NW
    ch = _sc_ch(epw)
    nchunk = epw // ch
    npt = n // _SC_NS
    assert n % _SC_NS == 0
    mesh = plsc.VectorSubcoreMesh(core_axis_name="c", subcore_axis_name="s")

    def body(mw_hbm, pm_hbm, dst2_hbm, zh_hbm, zp_hbm, agg2_hbm, dp2_hbm,
             idx2, mbuf, pbuf, acc_h, acc_p, sems):
        cid = jax.lax.axis_index("c")
        sid = jax.lax.axis_index("s")
        wid = sid * _SC_NC + cid
        base = wid * epw
        rows = pl.ds(sid * npt, npt)
        pltpu.sync_copy(zh_hbm.at[rows], acc_h.at[rows])
        pltpu.sync_copy(zp_hbm.at[rows], acc_p.at[rows])
        pltpu.sync_copy(dst2_hbm.at[pl.ds(wid * nchunk, nchunk)], idx2)
        plsc.subcore_barrier()

        def issue(j, b):
            off = pl.multiple_of(base + j * ch, 8)
            pltpu.async_copy(mw_hbm.at[pl.ds(off, ch)], mbuf.at[b], sems.at[b])
            pltpu.async_copy(pm_hbm.at[pl.ds(off, ch)], pbuf.at[b], sems.at[b])

        def process(j, b):
            pltpu.make_async_copy(mw_hbm.at[pl.ds(0, ch)], mbuf.at[b],
                                  sems.at[b]).wait()
            pltpu.make_async_copy(pm_hbm.at[pl.ds(0, ch)], pbuf.at[b],
                                  sems.at[b]).wait()
            pltpu.sync_copy(mbuf.at[b], acc_h.at[idx2.at[j]], add=True)
            pltpu.sync_copy(pbuf.at[b], acc_p.at[idx2.at[j]], add=True)

        issue(0, 0)

        def pair(i, carry):
            j0 = i * 2
            issue(j0 + 1, 1)
            process(j0, 0)
            issue(j0 + 2, 0)
            process(j0 + 1, 1)
            return carry

        jax.lax.fori_loop(0, (nchunk - 1) // 2, pair, 0, unroll=False)
        process(nchunk - 1, 0)
        plsc.subcore_barrier()

        @pl.when(cid == 0)
        def _():
            pltpu.sync_copy(acc_h.at[rows], agg2_hbm.at[0].at[rows])
            pltpu.sync_copy(acc_p.at[rows], dp2_hbm.at[0].at[rows])

        @pl.when(cid == 1)
        def _():
            pltpu.sync_copy(acc_h.at[rows], agg2_hbm.at[1].at[rows])
            pltpu.sync_copy(acc_p.at[rows], dp2_hbm.at[1].at[rows])

    f = pl.kernel(
        body,
        out_type=[
            jax.ShapeDtypeStruct((2, n, hd), _F32),
            jax.ShapeDtypeStruct((2, n, 16), _F32),
        ],
        mesh=mesh,
        compiler_params=pltpu.CompilerParams(use_tc_tiling_on_sc=False),
        scratch_types=[
            pltpu.VMEM((nchunk, ch), jnp.int32),
            pltpu.VMEM((2, ch, hd), _F32),
            pltpu.VMEM((2, ch, 16), _F32),
            pltpu.VMEM_SHARED((n, hd), _F32),
            pltpu.VMEM_SHARED((n, 16), _F32),
            pltpu.SemaphoreType.DMA((2,)),
        ],
    )
    return f(mw, pm, dst.reshape(e // ch, ch), zh, zp)


def _tables_body(h_ref, wa_ref, wb_ref, a_ref, b_ref):
    hv = h_ref[...]
    a_ref[...] = _mm(hv, wa_ref[...])
    b_ref[...] = _mm(hv, wb_ref[...])


def _tables_call(h, wa, wb):
    n, hd = h.shape
    nb = _blk(n, 2500)
    grid = (n // nb,)
    return pl.pallas_call(
        _tables_body,
        grid=grid,
        in_specs=[
            pl.BlockSpec((nb, hd), lambda i: (i, 0)),
            pl.BlockSpec((hd, hd), lambda i: (0, 0)),
            pl.BlockSpec((hd, hd), lambda i: (0, 0)),
        ],
        out_specs=[
            pl.BlockSpec((nb, hd), lambda i: (i, 0)),
            pl.BlockSpec((nb, hd), lambda i: (i, 0)),
        ],
        out_shape=[
            jax.ShapeDtypeStruct((n, hd), _F32),
            jax.ShapeDtypeStruct((n, hd), _F32),
        ],
    )(h, wa, wb)


def _edge_body(nfreq, esum_ref, relp_ref, w1c_ref, b1_ref, w2_ref, b2_ref,
               cw1_ref, cb1_ref, cw2_ref, mw_ref, pm_ref, loss_ref):
    rel = relp_ref[...]
    d2 = jnp.sum(rel * rel, axis=1, keepdims=True) + 1e-8
    dist = jnp.sqrt(d2)
    dist_t = jnp.transpose(dist)
    freqs_c = ((jax.lax.broadcasted_iota(jnp.int32, (nfreq, 1), 0).astype(_F32))
               + 1.0) * (math.pi / _CUTOFF)
    demb_t = jnp.sin(freqs_c * dist_t)
    pre = esum_ref[...] + jax.lax.dot_general(
        demb_t, w1c_ref[...], (((0,), (0,)), ((), ())),
        preferred_element_type=_F32) + b1_ref[...]
    m = _silu(pre)
    m = _silu(_mm(m, w2_ref[...]) + b2_ref[...])
    w_t = (0.5 * (jnp.cos(dist_t * (math.pi / _CUTOFF)) + 1.0)
           * (dist_t < _CUTOFF).astype(_F32))
    w = jnp.transpose(w_t)
    mw = m * w
    ch = _silu(_mm(mw, cw1_ref[...]) + cb1_ref[...])
    cs = jnp.tanh(_mm(ch, cw2_ref[...]))
    mw_ref[...] = mw
    pm_ref[...] = rel * (cs / (dist + 1.0))
    part = jnp.sum((dist_t - 1.5) ** 2 * w_t)

    @pl.when(pl.program_id(0) == 0)
    def _():
        loss_ref[0, 0] = part

    @pl.when(pl.program_id(0) != 0)
    def _():
        loss_ref[0, 0] += part


def _edge_call(esum, relp, w1c, b1, w2, b2, cw1, cb1, cw2):
    e, hd = esum.shape
    nfreq = w1c.shape[0]
    eb = _blk(e, 2500)
    grid = (e // eb,)
    import functools
    return pl.pallas_call(
        functools.partial(_edge_body, nfreq),
        grid=grid,
        in_specs=[
            pl.BlockSpec((eb, hd), lambda i: (i, 0)),
            pl.BlockSpec((eb, 16), lambda i: (i, 0)),
            pl.BlockSpec((nfreq, hd), lambda i: (0, 0)),
            pl.BlockSpec((1, hd), lambda i: (0, 0)),
            pl.BlockSpec((hd, hd), lambda i: (0, 0)),
            pl.BlockSpec((1, hd), lambda i: (0, 0)),
            pl.BlockSpec((hd, hd), lambda i: (0, 0)),
            pl.BlockSpec((1, hd), lambda i: (0, 0)),
            pl.BlockSpec((hd, 1), lambda i: (0, 0)),
        ],
        out_specs=[
            pl.BlockSpec((eb, hd), lambda i: (i, 0)),
            pl.BlockSpec((eb, 16), lambda i: (i, 0)),
            pl.BlockSpec(memory_space=pltpu.SMEM),
        ],
        out_shape=[
            jax.ShapeDtypeStruct((e, hd), _F32),
            jax.ShapeDtypeStruct((e, 16), _F32),
            jax.ShapeDtypeStruct((1, 1), _F32),
        ],
    )(esum, relp, w1c, b1, w2, b2, cw1, cb1, cw2)


def _node_body(h_ref, agg2a_ref, agg2b_ref, dp2a_ref, dp2b_ref, pos_ref,
               nw1a_ref, nw1b_ref, nb1_ref, nw2_ref, nb2_ref, hn_ref, pn_ref):
    agg = (agg2a_ref[0] + agg2a_ref[1]) + (agg2b_ref[0] + agg2b_ref[1])
    upd = _silu(_mm(h_ref[...], nw1a_ref[...]) + _mm(agg, nw1b_ref[...]) + nb1_ref[...])
    hn_ref[...] = h_ref[...] + _mm(upd, nw2_ref[...]) + nb2_ref[...]
    pn_ref[...] = (pos_ref[...] + (dp2a_ref[0] + dp2a_ref[1])
                   + (dp2b_ref[0] + dp2b_ref[1]))


def _node_call(h, agg2a, agg2b, dp2a, dp2b, pos16, nw1a, nw1b, nb1, nw2, nb2):
    n, hd = h.shape
    nb = _blk(n, 2500)
    grid = (n // nb,)
    return pl.pallas_call(
        _node_body,
        grid=grid,
        in_specs=[
            pl.BlockSpec((nb, hd), lambda i: (i, 0)),
            pl.BlockSpec((2, nb, hd), lambda i: (0, i, 0)),
            pl.BlockSpec((2, nb, hd), lambda i: (0, i, 0)),
            pl.BlockSpec((2, nb, 16), lambda i: (0, i, 0)),
            pl.BlockSpec((2, nb, 16), lambda i: (0, i, 0)),
            pl.BlockSpec((nb, 16), lambda i: (i, 0)),
            pl.BlockSpec((hd, hd), lambda i: (0, 0)),
            pl.BlockSpec((hd, hd), lambda i: (0, 0)),
            pl.BlockSpec((1, hd), lambda i: (0, 0)),
            pl.BlockSpec((hd, hd), lambda i: (0, 0)),
            pl.BlockSpec((1, hd), lambda i: (0, 0)),
        ],
        out_specs=[
            pl.BlockSpec((nb, hd), lambda i: (i, 0)),
            pl.BlockSpec((nb, 16), lambda i: (i, 0)),
        ],
        out_shape=[
            jax.ShapeDtypeStruct((n, hd), _F32),
            jax.ShapeDtypeStruct((n, 16), _F32),
        ],
    )(h, agg2a, agg2b, dp2a, dp2b, pos16, nw1a, nw1b, nb1, nw2, nb2)


def _colsum_body(pos_ref, out_ref):
    s = jnp.sum(pos_ref[...], axis=0, keepdims=True)

    @pl.when(pl.program_id(0) == 0)
    def _():
        out_ref[...] = s

    @pl.when(pl.program_id(0) != 0)
    def _():
        out_ref[...] += s


def _colsum_call(pos16):
    n, _ = pos16.shape
    nb = _blk(n, 2500)
    return pl.pallas_call(
        _colsum_body,
        grid=(n // nb,),
        in_specs=[pl.BlockSpec((nb, 16), lambda i: (i, 0))],
        out_specs=pl.BlockSpec((1, 16), lambda i: (0, 0)),
        out_shape=jax.ShapeDtypeStruct((1, 16), _F32),
    )(pos16)


def _heads_body(n_nodes, h_ref, pos_ref, csum_ref, gw1_ref, gb1_ref, gw2_ref,
                gb2_ref, iw1a_ref, iw1b0_ref, wmean_ref, ib1_ref, iw2_ref,
                ib2_ref, geo_ref, inv_ref):
    hv = h_ref[...]
    geo_ref[...] = _mm(_silu(_mm(hv, gw1_ref[...]) + gb1_ref[...]), gw2_ref[...]) + gb2_ref[...]
    posv = pos_ref[...]
    norm = jnp.sqrt(jnp.sum(posv * posv, axis=1, keepdims=True))
    mean16 = csum_ref[...] * (1.0 / n_nodes)
    crow = _mm(mean16, wmean_ref[...])
    pre = _mm(hv, iw1a_ref[...]) + norm * iw1b0_ref[...] + crow + ib1_ref[...]
    inv_ref[...] = _mm(_silu(pre), iw2_ref[...]) + ib2_ref[...]


def _heads_call(h, pos16, csum, gw1, gb1, gw2, gb2, iw1a, iw1b0, wmean, ib1,
                iw2, ib2):
    n, hd = h.shape
    go = gw2.shape[1]
    nb = _blk(n, 2500)
    import functools
    return pl.pallas_call(
        functools.partial(_heads_body, n),
        grid=(n // nb,),
        in_specs=[
            pl.BlockSpec((nb, hd), lambda i: (i, 0)),
            pl.BlockSpec((nb, 16), lambda i: (i, 0)),
            pl.BlockSpec((1, 16), lambda i: (0, 0)),
            pl.BlockSpec((hd, gw1.shape[1]), lambda i: (0, 0)),
            pl.BlockSpec((1, gb1.shape[1]), lambda i: (0, 0)),
            pl.BlockSpec((gw2.shape[0], go), lambda i: (0, 0)),
            pl.BlockSpec((1, go), lambda i: (0, 0)),
            pl.BlockSpec((hd, hd), lambda i: (0, 0)),
            pl.BlockSpec((1, hd), lambda i: (0, 0)),
            pl.BlockSpec((16, hd), lambda i: (0, 0)),
            pl.BlockSpec((1, hd), lambda i: (0, 0)),
            pl.BlockSpec((hd, hd), lambda i: (0, 0)),
            pl.BlockSpec((1, hd), lambda i: (0, 0)),
        ],
        out_specs=[
            pl.BlockSpec((nb, go), lambda i: (i, 0)),
            pl.BlockSpec((nb, hd), lambda i: (i, 0)),
        ],
        out_shape=[
            jax.ShapeDtypeStruct((n, go), _F32),
            jax.ShapeDtypeStruct((n, hd), _F32),
        ],
    )(h, pos16, csum, gw1, gb1, gw2, gb2, iw1a, iw1b0, wmean, ib1, iw2, ib2)


def kernel(h, pos, batch, edge_index, params):
    n, hd = h.shape
    e = edge_index.shape[1]
    nl = 0
    while f"e{nl}_W1" in params:
        nl += 1
    src = edge_index[0]
    dst = edge_index[1]
    pos16 = jnp.zeros((n, 16), _F32).at[:, :3].set(pos)

    def b2d(v):
        return v.reshape(1, -1)

    loss_parts = []
    zh = jnp.zeros((n, hd), _F32)
    zp = jnp.zeros((n, 16), _F32)
    e2 = e // 2
    src0, src1 = src[:e2], src[e2:]
    dst0, dst1 = dst[:e2], dst[e2:]
    for l in range(nl):
        w1 = params[f"e{l}_W1"]
        eargs = (w1[2 * hd:], b2d(params[f"e{l}_b1"]),
                 params[f"e{l}_W2"], b2d(params[f"e{l}_b2"]),
                 params[f"c{l}_W1"], b2d(params[f"c{l}_b1"]),
                 params[f"c{l}_W2"])
        ta, tb = _tables_call(h, w1[:hd], w1[hd:2 * hd])
        es0, rp0 = _sc_gather(ta, tb, pos16, src0, dst0)
        es1, rp1 = _sc_gather(ta, tb, pos16, src1, dst1)
        mw0, pm0, lp0 = _edge_call(es0, rp0, *eargs)
        agg2a, dp2a = _sc_scatter(mw0, pm0, dst0, n, zh, zp)
        mw1, pm1, lp1 = _edge_call(es1, rp1, *eargs)
        agg2b, dp2b = _sc_scatter(mw1, pm1, dst1, n, zh, zp)
        nw1 = params[f"n{l}_W1"]
        h, pos16 = _node_call(
            h, agg2a, agg2b, dp2a, dp2b, pos16, nw1[:hd], nw1[hd:],
            b2d(params[f"n{l}_b1"]), params[f"n{l}_W2"], b2d(params[f"n{l}_b2"]))
        loss_parts.append(lp0[0, 0] + lp1[0, 0])

    csum = _colsum_call(pos16)
    iw1 = params["i_W1"]
    wmean = jnp.zeros((16, hd), _F32).at[:3].set(iw1[hd + 1:hd + 4])
    geo, inv = _heads_call(
        h, pos16, csum, params["g_W1"], b2d(params["g_b1"]), params["g_W2"],
        b2d(params["g_b2"]), iw1[:hd], iw1[hd:hd + 1], wmean,
        b2d(params["i_b1"]), params["i_W2"], b2d(params["i_b2"]))

    closs = sum(loss_parts) / e
    return (h, pos16[:, :3], geo, inv, closs)
